# baseline (speedup 1.0000x reference)
import functools

import jax
import jax.numpy as jnp
from jax import lax
from jax.experimental import pallas as pl
from jax.experimental.pallas import tpu as pltpu
from jax.experimental.pallas import tpu_sc as plsc

_BATCH = 16384
_HIST = 50
_DIM = 32
_B = _BATCH * _HIST
_W = 64
_NBLK = _B // _W
_NC = 2
_NS = 16
_NWORK = _NC * _NS
_BLK_PW = _NBLK // _NWORK
_K = 25
_NROUND = _BLK_PW // _K
_RND_ROWS = _K * _W


def _tec_body(idx_hbm, table_hbm, out_hbm, idx_v, rows_v,
              isem0, isem1, gsem0, gsem1, osem0, osem1):
    wid = lax.axis_index("s") * _NC + lax.axis_index("c")
    base = wid * _BLK_PW
    rbase = wid * _BLK_PW * _W
    isems = (isem0, isem1)
    gsems = (gsem0, gsem1)
    osems = (osem0, osem1)

    def idx_copy(b, r):
        return pltpu.make_async_copy(
            idx_hbm.at[pl.ds(base + r * _K, _K)], idx_v.at[b], isems[b])

    def fire_gathers(b):
        def issue(j, carry):
            pltpu.make_async_copy(
                table_hbm.at[idx_v.at[b, j]],
                rows_v.at[b, pl.ds(j * _W, _W)],
                gsems[b],
            ).start()
            return carry
        lax.fori_loop(0, _K, issue, 0)

    def drain_gathers(b):
        pltpu.make_async_copy(
            table_hbm.at[pl.ds(0, _RND_ROWS)], rows_v.at[b], gsems[b]).wait()

    def out_copy(b, r):
        return pltpu.make_async_copy(
            rows_v.at[b], out_hbm.at[pl.ds(rbase + r * _RND_ROWS, _RND_ROWS)],
            osems[b])

    idx_copy(0, 0).start()
    idx_copy(1, 1).start()
    idx_copy(0, 0).wait()
    fire_gathers(0)

    def step(g, carry):
        for b in range(2):
            r = 2 * g + b
            b2 = 1 - b

            @pl.when(r >= 1)
            def _():
                out_copy(b2, r - 1).wait()

            @pl.when(r + 1 <= _NROUND - 1)
            def _():
                idx_copy(b2, r + 1).wait()
                fire_gathers(b2)

            drain_gathers(b)
            out_copy(b, r).start()

            @pl.when(r + 2 <= _NROUND - 1)
            def _():
                idx_copy(b, r + 2).start()
        return carry

    lax.fori_loop(0, _NROUND // 2, step, 0)

    out_copy(1, _NROUND - 1).wait()


@jax.jit
def _gather(item_ids_blocked, table):
    mesh = plsc.VectorSubcoreMesh(core_axis_name="c", subcore_axis_name="s")
    fn = functools.partial(
        pl.kernel,
        mesh=mesh,
        out_type=jax.ShapeDtypeStruct((_B, _DIM), jnp.float32),
        scratch_types=[
            pltpu.VMEM((2, _K, _W), jnp.int32),
            pltpu.VMEM((2, _RND_ROWS, _DIM), jnp.float32),
            pltpu.SemaphoreType.DMA,
            pltpu.SemaphoreType.DMA,
            pltpu.SemaphoreType.DMA,
            pltpu.SemaphoreType.DMA,
            pltpu.SemaphoreType.DMA,
            pltpu.SemaphoreType.DMA,
        ],
        compiler_params=pltpu.CompilerParams(use_tc_tiling_on_sc=False),
    )(_tec_body)
    return fn(item_ids_blocked, table)


def kernel(item_ids, table):
    ids_blocked = item_ids.reshape(_NBLK, _W)
    out = _gather(ids_blocked, table)
    return out.reshape(_BATCH, _HIST, _DIM)


# speedup vs baseline: 1.6254x; 1.6254x over previous
import functools

import jax
import jax.numpy as jnp
from jax import lax
from jax.experimental import pallas as pl
from jax.experimental.pallas import tpu as pltpu
from jax.experimental.pallas import tpu_sc as plsc

_BATCH = 16384
_HIST = 50
_DIM = 32
_B = _BATCH * _HIST
_W = 64
_NBLK = _B // _W
_NC = 2
_NS = 16
_NWORK = _NC * _NS
_BLK_PW = _NBLK // _NWORK
_K = 25
_NROUND = _BLK_PW // _K


def _tec_body(idx_hbm, table_hbm, out_hbm, idx_v, rows_v,
              isem0, isem1, gsem0, gsem1, osem0, osem1):
    wid = lax.axis_index("s") * _NC + lax.axis_index("c")
    base = wid * _BLK_PW
    isems = (isem0, isem1)
    gsems = (gsem0, gsem1)
    osems = (osem0, osem1)

    def idx_copy(b, r):
        return pltpu.make_async_copy(
            idx_hbm.at[pl.ds(base + r * _K, _K)], idx_v.at[b], isems[b])

    def gather(b, j):
        return pltpu.make_async_copy(
            table_hbm.at[idx_v.at[b].at[j]], rows_v.at[b].at[j], gsems[b])

    def out_copy(b, r):
        return pltpu.make_async_copy(
            rows_v.at[b], out_hbm.at[pl.ds(base + r * _K, _K)], osems[b])

    idx_copy(0, 0).start()
    idx_copy(1, 1).start()
    idx_copy(0, 0).wait()
    for j in range(_K):
        gather(0, j).start()

    def step(g, carry):
        for b in range(2):
            r = 2 * g + b
            b2 = 1 - b

            @pl.when(r >= 1)
            def _():
                out_copy(b2, r - 1).wait()

            @pl.when(r + 1 <= _NROUND - 1)
            def _():
                idx_copy(b2, r + 1).wait()
                for j in range(_K):
                    gather(b2, j).start()

            for j in range(_K):
                gather(b, j).wait()
            out_copy(b, r).start()

            @pl.when(r + 2 <= _NROUND - 1)
            def _():
                idx_copy(b, r + 2).start()
        return carry

    lax.fori_loop(0, _NROUND // 2, step, 0)

    out_copy(1, _NROUND - 1).wait()


@jax.jit
def _gather(item_ids_blocked, table):
    mesh = plsc.VectorSubcoreMesh(core_axis_name="c", subcore_axis_name="s")
    fn = functools.partial(
        pl.kernel,
        mesh=mesh,
        out_type=jax.ShapeDtypeStruct((_NBLK, _W, _DIM), jnp.float32),
        scratch_types=[
            pltpu.VMEM((2, _K, _W), jnp.int32),
            pltpu.VMEM((2, _K, _W, _DIM), jnp.float32),
            pltpu.SemaphoreType.DMA,
            pltpu.SemaphoreType.DMA,
            pltpu.SemaphoreType.DMA,
            pltpu.SemaphoreType.DMA,
            pltpu.SemaphoreType.DMA,
            pltpu.SemaphoreType.DMA,
        ],
        compiler_params=pltpu.CompilerParams(use_tc_tiling_on_sc=False),
    )(_tec_body)
    return fn(item_ids_blocked, table)


def kernel(item_ids, table):
    ids_blocked = item_ids.reshape(_NBLK, _W)
    out = _gather(ids_blocked, table)
    return out.reshape(_BATCH, _HIST, _DIM)
